# baseline (speedup 1.0000x reference)
import functools

import jax
import jax.numpy as jnp
from jax import lax
from jax.experimental import pallas as pl
from jax.experimental.pallas import tpu as pltpu
from jax.experimental.pallas import tpu_sc as plsc

_N_VARS = 100000
_N_TERMS = 16384
_B = 1024

_N_TILES = 32
_T_PER_TILE = _N_TERMS // _N_TILES
_CHUNK = 16
_N_CHUNKS = _T_PER_TILE // _CHUNK


@functools.lru_cache(maxsize=None)
def _colsum_fn():
    mesh = plsc.VectorSubcoreMesh(core_axis_name="c", subcore_axis_name="s")

    @functools.partial(
        pl.kernel,
        out_type=jax.ShapeDtypeStruct((2, _B), jnp.float32),
        mesh=mesh,
        scratch_types=[
            pltpu.VMEM((_T_PER_TILE,), jnp.int32),
            pltpu.VMEM((_T_PER_TILE, 16), jnp.float32),
            pltpu.VMEM((_CHUNK, _B), jnp.float32),
            pltpu.VMEM((_CHUNK, _B), jnp.float32),
            pltpu.VMEM((_B,), jnp.float32),
            pltpu.VMEM((4, _B), jnp.float32),
            pltpu.VMEM((_B,), jnp.float32),
            pltpu.VMEM_SHARED((16, _B), jnp.float32),
            pltpu.SemaphoreType.DMA,
            pltpu.SemaphoreType.DMA,
        ],
        compiler_params=pltpu.CompilerParams(needs_layout_passes=False),
    )
    def _colsum(y_hbm, idx_hbm, cb_hbm, o_hbm,
                idx_v, cb_v, buf0, buf1, acc_v, sum_v, red_v, spacc,
                sem0, sem1):
        cid = lax.axis_index("c")
        sid = lax.axis_index("s")
        wid = sid * 2 + cid
        base_t = wid * _T_PER_TILE

        pltpu.sync_copy(idx_hbm.at[pl.ds(base_t, _T_PER_TILE)], idx_v)
        pltpu.sync_copy(cb_hbm.at[pl.ds(base_t, _T_PER_TILE)], cb_v)

        def zero_body(r, carry):
            acc_v[pl.ds(r * 16, 16)] = jnp.zeros((16,), jnp.float32)
            return carry

        lax.fori_loop(0, _B // 16, zero_body, 0)

        bufs = (buf0, buf1)
        sems = (sem0, sem1)

        def start(chunk, b):
            return pltpu.async_copy(
                y_hbm.at[idx_v.at[pl.ds(chunk * _CHUNK, _CHUNK)]],
                bufs[b], sems[b])

        def wait(b):
            pltpu.make_async_copy(
                y_hbm.at[idx_v.at[pl.ds(0, _CHUNK)]], bufs[b], sems[b],
            ).wait()

        def consume(chunk, buf):
            cvecs = [cb_v[chunk * _CHUNK + j, :] for j in range(_CHUNK)]

            def row_body(r, carry):
                for u in range(4):
                    off = r * 64 + u * 16
                    a = acc_v[pl.ds(off, 16)]
                    for j in range(_CHUNK):
                        a = a + buf[j, pl.ds(off, 16)] * cvecs[j]
                    acc_v[pl.ds(off, 16)] = a
                return carry

            lax.fori_loop(0, _B // 64, row_body, 0)

        start(0, 0)
        start(1, 1)

        def pair_body(p, carry):
            chunk = p * 2
            for b in range(2):
                wait(b)
                consume(chunk + b, bufs[b])
                nxt = chunk + b + 2

                @pl.when(nxt < _N_CHUNKS)
                def _():
                    start(nxt, b)
            return carry

        lax.fori_loop(0, _N_CHUNKS // 2, pair_body, 0)

        pltpu.sync_copy(acc_v, spacc.at[sid])
        plsc.subcore_barrier()

        @pl.when(sid == 0)
        def _():
            def zred_body(r, carry):
                red_v[pl.ds(r * 16, 16)] = jnp.zeros((16,), jnp.float32)
                return carry

            lax.fori_loop(0, _B // 16, zred_body, 0)
            for g in range(4):
                pltpu.sync_copy(spacc.at[pl.ds(g * 4, 4)], sum_v)

                def red_body(r, carry):
                    def tsum(t, v):
                        return v + sum_v[t, pl.ds(r * 16, 16)]

                    v = lax.fori_loop(0, 4, tsum,
                                      jnp.zeros((16,), jnp.float32))
                    plsc.addupdate(red_v.at[pl.ds(r * 16, 16)], v)
                    return carry

                lax.fori_loop(0, _B // 16, red_body, 0)
            pltpu.sync_copy(red_v, o_hbm.at[cid])

    return _colsum


def _combine_body(p_ref, o_ref):
    lhs = jnp.sum(p_ref[...], axis=0, keepdims=True)
    o_ref[...] = (lhs <= 0.0).astype(jnp.int32)


def _combine_compare(partials):
    return pl.pallas_call(
        _combine_body,
        out_shape=jax.ShapeDtypeStruct((1, _B), jnp.int32),
    )(partials)


def kernel(x, indices_tensor, coeff_tensor):
    y = x.T
    order = jnp.argsort(indices_tensor)
    idx = indices_tensor[order].astype(jnp.int32)
    cb = jnp.broadcast_to(
        coeff_tensor[order].astype(jnp.float32)[:, None], (_N_TERMS, 16))
    partials = _colsum_fn()(y, idx, cb)
    out = _combine_compare(partials)
    return out.reshape(_B).astype(jnp.bool_)


# speedup vs baseline: 1.2949x; 1.2949x over previous
import functools

import jax
import jax.numpy as jnp
from jax import lax
from jax.experimental import pallas as pl
from jax.experimental.pallas import tpu as pltpu
from jax.experimental.pallas import tpu_sc as plsc

_N_VARS = 100000
_N_TERMS = 16384
_B = 1024

_N_TILES = 32
_T_PER_TILE = _N_TERMS // _N_TILES
_CHUNK = 32
_N_CHUNKS = _T_PER_TILE // _CHUNK


@functools.lru_cache(maxsize=None)
def _colsum_fn():
    mesh = plsc.VectorSubcoreMesh(core_axis_name="c", subcore_axis_name="s")

    @functools.partial(
        pl.kernel,
        out_type=jax.ShapeDtypeStruct((2, _B), jnp.float32),
        mesh=mesh,
        scratch_types=[
            pltpu.VMEM((_T_PER_TILE,), jnp.int32),
            pltpu.VMEM((_T_PER_TILE,), jnp.float32),
            pltpu.VMEM((_CHUNK, _B), jnp.float32),
            pltpu.VMEM((_CHUNK, _B), jnp.float32),
            pltpu.VMEM((_B,), jnp.float32),
            pltpu.VMEM((4, _B), jnp.float32),
            pltpu.VMEM((_B,), jnp.float32),
            pltpu.VMEM_SHARED((16, _B), jnp.float32),
            pltpu.SemaphoreType.DMA,
            pltpu.SemaphoreType.DMA,
        ],
        compiler_params=pltpu.CompilerParams(needs_layout_passes=False),
    )
    def _colsum(y_hbm, idx_hbm, cb_hbm, o_hbm,
                idx_v, cb_v, buf0, buf1, acc_v, sum_v, red_v, spacc,
                sem0, sem1):
        cid = lax.axis_index("c")
        sid = lax.axis_index("s")
        wid = sid * 2 + cid
        base_t = wid * _T_PER_TILE

        pltpu.sync_copy(idx_hbm.at[pl.ds(base_t, _T_PER_TILE)], idx_v)
        pltpu.sync_copy(cb_hbm.at[pl.ds(base_t, _T_PER_TILE)], cb_v)

        def zero_body(r, carry):
            acc_v[pl.ds(r * 16, 16)] = jnp.zeros((16,), jnp.float32)
            return carry

        lax.fori_loop(0, _B // 16, zero_body, 0)

        bufs = (buf0, buf1)
        sems = (sem0, sem1)

        def start(chunk, b):
            return pltpu.async_copy(
                y_hbm.at[idx_v.at[pl.ds(chunk * _CHUNK, _CHUNK)]],
                bufs[b], sems[b])

        def wait(b):
            pltpu.make_async_copy(
                y_hbm.at[idx_v.at[pl.ds(0, _CHUNK)]], bufs[b], sems[b],
            ).wait()

        def consume(chunk, buf):
            cvecs = [
                plsc.load_gather(
                    cb_v, [jnp.full((16,), chunk * _CHUNK + j, jnp.int32)])
                for j in range(_CHUNK)
            ]

            def row_body(r, carry):
                for u in range(4):
                    off = r * 64 + u * 16
                    a = acc_v[pl.ds(off, 16)]
                    for j in range(_CHUNK):
                        a = a + buf[j, pl.ds(off, 16)] * cvecs[j]
                    acc_v[pl.ds(off, 16)] = a
                return carry

            lax.fori_loop(0, _B // 64, row_body, 0)

        start(0, 0)
        start(1, 1)

        def pair_body(p, carry):
            chunk = p * 2
            for b in range(2):
                wait(b)
                consume(chunk + b, bufs[b])
                nxt = chunk + b + 2

                @pl.when(nxt < _N_CHUNKS)
                def _():
                    start(nxt, b)
            return carry

        lax.fori_loop(0, _N_CHUNKS // 2, pair_body, 0)

        pltpu.sync_copy(acc_v, spacc.at[sid])
        plsc.subcore_barrier()

        @pl.when(sid == 0)
        def _():
            def zred_body(r, carry):
                red_v[pl.ds(r * 16, 16)] = jnp.zeros((16,), jnp.float32)
                return carry

            lax.fori_loop(0, _B // 16, zred_body, 0)
            for g in range(4):
                pltpu.sync_copy(spacc.at[pl.ds(g * 4, 4)], sum_v)

                def red_body(r, carry):
                    def tsum(t, v):
                        return v + sum_v[t, pl.ds(r * 16, 16)]

                    v = lax.fori_loop(0, 4, tsum,
                                      jnp.zeros((16,), jnp.float32))
                    plsc.addupdate(red_v.at[pl.ds(r * 16, 16)], v)
                    return carry

                lax.fori_loop(0, _B // 16, red_body, 0)
            pltpu.sync_copy(red_v, o_hbm.at[cid])

    return _colsum


def _combine_body(p_ref, o_ref):
    lhs = jnp.sum(p_ref[...], axis=0, keepdims=True)
    o_ref[...] = (lhs <= 0.0).astype(jnp.int32)


def _combine_compare(partials):
    return pl.pallas_call(
        _combine_body,
        out_shape=jax.ShapeDtypeStruct((1, _B), jnp.int32),
    )(partials)


def kernel(x, indices_tensor, coeff_tensor):
    y = x.T
    idx = indices_tensor.astype(jnp.int32)
    cb = coeff_tensor.astype(jnp.float32)
    partials = _colsum_fn()(y, idx, cb)
    out = _combine_compare(partials)
    return out.reshape(_B).astype(jnp.bool_)
